# baseline (speedup 1.0000x reference)
import jax
import jax.numpy as jnp
from jax.experimental import pallas as pl
from jax.experimental.pallas import tpu as pltpu

_BLK = 512


def _split_body(x_ref, o1_ref, o2_ref):
    blk, H, W = x_ref.shape
    x3 = x_ref[...]
    o1_ref[...] = x3[:, : H // 2, :].reshape(blk, H, W // 2)
    o2_ref[...] = x3[:, H // 2 :, :].reshape(blk, H, W // 2)


def kernel(x):
    B, C, H, W = x.shape
    rows = B * C
    xr = x.reshape(rows, H, W)
    grid = rows // _BLK
    o1, o2 = pl.pallas_call(
        _split_body,
        grid=(grid,),
        in_specs=[pl.BlockSpec((_BLK, H, W), lambda i: (i, 0, 0))],
        out_specs=[
            pl.BlockSpec((_BLK, H, W // 2), lambda i: (i, 0, 0)),
            pl.BlockSpec((_BLK, H, W // 2), lambda i: (i, 0, 0)),
        ],
        out_shape=[
            jax.ShapeDtypeStruct((rows, H, W // 2), x.dtype),
            jax.ShapeDtypeStruct((rows, H, W // 2), x.dtype),
        ],
    )(xr)
    X1 = o1.reshape(B, C, H, W // 2)
    X2 = o2.reshape(B, C, H, W // 2)
    return (X1, X2)


# speedup vs baseline: 2.0111x; 2.0111x over previous
import jax
import jax.numpy as jnp
from jax.experimental import pallas as pl
from jax.experimental.pallas import tpu as pltpu

_BLK = 1024


def _split_body(x_ref, o1_ref, o2_ref):
    o1_ref[...] = x_ref[:, 0]
    o2_ref[...] = x_ref[:, 1]


def kernel(x):
    B, C, H, W = x.shape
    rows = B * C
    xv = x.reshape(rows, 2, 4, 128)
    grid = rows // _BLK
    o1, o2 = pl.pallas_call(
        _split_body,
        grid=(grid,),
        in_specs=[pl.BlockSpec((_BLK, 2, 4, 128), lambda i: (i, 0, 0, 0))],
        out_specs=[
            pl.BlockSpec((_BLK, 4, 128), lambda i: (i, 0, 0)),
            pl.BlockSpec((_BLK, 4, 128), lambda i: (i, 0, 0)),
        ],
        out_shape=[
            jax.ShapeDtypeStruct((rows, 4, 128), x.dtype),
            jax.ShapeDtypeStruct((rows, 4, 128), x.dtype),
        ],
    )(xv)
    X1 = o1.reshape(B, C, H, W // 2)
    X2 = o2.reshape(B, C, H, W // 2)
    return (X1, X2)


# speedup vs baseline: 20.7604x; 10.3229x over previous
import jax
import jax.numpy as jnp
from jax.experimental import pallas as pl
from jax.experimental.pallas import tpu as pltpu


def _split_body(x_ref, o1_ref, o2_ref):
    nt = o1_ref.shape[1]
    o1_ref[...] = x_ref[:, :nt, :]
    o2_ref[...] = x_ref[:, nt:, :]


def kernel(x):
    B, C, H, W = x.shape
    HW = H * W
    nt = HW // 2
    y = jnp.transpose(x, (0, 2, 3, 1)).reshape(B, HW, C)
    o1, o2 = pl.pallas_call(
        _split_body,
        grid=(B,),
        in_specs=[pl.BlockSpec((1, HW, C), lambda i: (i, 0, 0))],
        out_specs=[
            pl.BlockSpec((1, nt, C), lambda i: (i, 0, 0)),
            pl.BlockSpec((1, nt, C), lambda i: (i, 0, 0)),
        ],
        out_shape=[
            jax.ShapeDtypeStruct((B, nt, C), x.dtype),
            jax.ShapeDtypeStruct((B, nt, C), x.dtype),
        ],
    )(y)
    X1 = o1.reshape(B, H, nt // W, C).transpose(0, 3, 1, 2)
    X2 = o2.reshape(B, H, nt // W, C).transpose(0, 3, 1, 2)
    return (X1, X2)
